# baseline (speedup 1.0000x reference)
import functools
import jax
import jax.numpy as jnp
from jax import lax
from jax.experimental import pallas as pl
from jax.experimental.pallas import tpu as pltpu
from jax.experimental.pallas import tpu_sc as plsc

D_MODEL = 768
D_FF = 2048
NE = 8
TOPK = 2
T = 2048
NP = T * TOPK
BT = 256
N_TILES = NP // BT + NE
N_MAX = N_TILES * BT
NW = 32
GPW = N_MAX // NW
GCH = 64
TPW = T // NW
CCH = 32


def _ffn_tile(te_ref, nl_ref, xs_ref, g_ref, w1_ref, w3_ref, w2_ref, out_ref):
    i = pl.program_id(0)

    @pl.when(i < nl_ref[0])
    def _():
        xv = xs_ref[...]
        h = jnp.dot(xv, w1_ref[0], preferred_element_type=jnp.float32)
        g = jnp.dot(xv, w3_ref[0], preferred_element_type=jnp.float32)
        a = (h * jax.nn.sigmoid(h)) * g
        y = jnp.dot(a, w2_ref[0], preferred_element_type=jnp.float32)
        out_ref[...] = y * g_ref[0, 0][:, None]


def _ffn(te, nl, xs, gate_tiles, W1, W3, W2):
    grid_spec = pltpu.PrefetchScalarGridSpec(
        num_scalar_prefetch=2,
        grid=(N_TILES,),
        in_specs=[
            pl.BlockSpec((BT, D_MODEL), lambda i, te, nl: (i, 0)),
            pl.BlockSpec((1, 1, BT), lambda i, te, nl: (i, 0, 0)),
            pl.BlockSpec((1, D_MODEL, D_FF), lambda i, te, nl: (te[i], 0, 0)),
            pl.BlockSpec((1, D_MODEL, D_FF), lambda i, te, nl: (te[i], 0, 0)),
            pl.BlockSpec((1, D_FF, D_MODEL), lambda i, te, nl: (te[i], 0, 0)),
        ],
        out_specs=pl.BlockSpec((BT, D_MODEL), lambda i, te, nl: (i, 0)),
    )
    return pl.pallas_call(
        _ffn_tile,
        grid_spec=grid_spec,
        out_shape=jax.ShapeDtypeStruct((N_MAX, D_MODEL), jnp.float32),
        compiler_params=pltpu.CompilerParams(
            dimension_semantics=("arbitrary",)),
    )(te, nl, xs, gate_tiles, W1, W3, W2)


@functools.cache
def _sc_mesh():
    return plsc.VectorSubcoreMesh(
        core_axis_name="c", subcore_axis_name="s", num_cores=2)


def _gather_rows_body(x_hbm, idx_hbm, out_hbm, idx_v, rows_a, rows_b, sem_a, sem_b):
    wid = lax.axis_index("s") * 2 + lax.axis_index("c")
    base = wid * GPW
    pltpu.sync_copy(idx_hbm.at[pl.ds(base, GPW)], idx_v)
    bufs = (rows_a, rows_b)
    sems = (sem_a, sem_b)
    nch = GPW // GCH
    cps = [None, None]
    cps[0] = pltpu.async_copy(
        x_hbm.at[idx_v.at[pl.ds(0, GCH)]], rows_a, sem_a)
    for c in range(nch):
        if c + 1 < nch:
            cps[(c + 1) % 2] = pltpu.async_copy(
                x_hbm.at[idx_v.at[pl.ds((c + 1) * GCH, GCH)]],
                bufs[(c + 1) % 2], sems[(c + 1) % 2])
        cps[c % 2].wait()
        pltpu.sync_copy(bufs[c % 2], out_hbm.at[pl.ds(base + c * GCH, GCH)])


@functools.cache
def _gather_rows():
    return functools.partial(
        pl.kernel, mesh=_sc_mesh(),
        out_type=jax.ShapeDtypeStruct((N_MAX, D_MODEL), jnp.float32),
        scratch_types=[
            pltpu.VMEM((GPW,), jnp.int32),
            pltpu.VMEM((GCH, D_MODEL), jnp.float32),
            pltpu.VMEM((GCH, D_MODEL), jnp.float32),
            pltpu.SemaphoreType.DMA,
            pltpu.SemaphoreType.DMA,
        ],
    )(_gather_rows_body)


def _combine_rows_body(ys_hbm, p0_hbm, p1_hbm, out_hbm, i0_v, i1_v, r0_v, r1_v, s0, s1):
    wid = lax.axis_index("s") * 2 + lax.axis_index("c")
    base = wid * TPW

    def chunk(c, carry):
        off = base + c * CCH
        pltpu.sync_copy(p0_hbm.at[pl.ds(off, CCH)], i0_v)
        pltpu.sync_copy(p1_hbm.at[pl.ds(off, CCH)], i1_v)
        cp0 = pltpu.async_copy(ys_hbm.at[i0_v], r0_v, s0)
        cp1 = pltpu.async_copy(ys_hbm.at[i1_v], r1_v, s1)
        cp0.wait()
        cp1.wait()

        def row(i, rc):
            for j in range(D_MODEL // 16):
                sl = pl.ds(j * 16, 16)
                r0_v[i, sl] = r0_v[i, sl] + r1_v[i, sl]
            return rc

        lax.fori_loop(0, CCH, row, 0)
        pltpu.sync_copy(r0_v, out_hbm.at[pl.ds(off, CCH)])
        return carry

    lax.fori_loop(0, TPW // CCH, chunk, 0)


@functools.cache
def _combine_rows():
    return functools.partial(
        pl.kernel, mesh=_sc_mesh(),
        out_type=jax.ShapeDtypeStruct((T, D_MODEL), jnp.float32),
        scratch_types=[
            pltpu.VMEM((CCH,), jnp.int32),
            pltpu.VMEM((CCH,), jnp.int32),
            pltpu.VMEM((CCH, D_MODEL), jnp.float32),
            pltpu.VMEM((CCH, D_MODEL), jnp.float32),
            pltpu.SemaphoreType.DMA,
            pltpu.SemaphoreType.DMA,
        ],
    )(_combine_rows_body)


def kernel(x, Wg, W1, W3, W2):
    b, s, d = x.shape
    xf = x.reshape(-1, d)

    logits = xf @ Wg
    gates = jax.nn.softmax(logits, axis=-1)
    tg, ti = lax.top_k(gates, TOPK)
    tg = tg / jnp.sum(tg, axis=-1, keepdims=True)

    eid = ti.reshape(-1).astype(jnp.int32)
    oh = (eid[:, None] == jnp.arange(NE, dtype=jnp.int32)[None, :]).astype(jnp.int32)
    within = jnp.cumsum(oh, axis=0) - oh
    rank = jnp.take_along_axis(within, eid[:, None], axis=1)[:, 0]
    counts = jnp.sum(oh, axis=0)
    padded = ((counts + BT - 1) // BT) * BT
    cumpad = jnp.cumsum(padded)
    offs = cumpad - padded
    dst = offs[eid] + rank
    total = cumpad[-1]
    n_live = total // BT

    tile_starts = jnp.arange(N_TILES, dtype=jnp.int32) * BT
    te = jnp.searchsorted(cumpad, tile_starts, side="right").astype(jnp.int32)
    te_lastlive = jnp.take(te, jnp.maximum(n_live - 1, 0))
    te = jnp.where(tile_starts < total, te, te_lastlive)

    row_token = jnp.zeros((N_MAX,), jnp.int32).at[dst].set(
        jnp.arange(NP, dtype=jnp.int32) // TOPK)
    row_gate = jnp.zeros((N_MAX,), jnp.float32).at[dst].set(tg.reshape(-1))
    gate_tiles = row_gate.reshape(N_TILES, 1, BT)
    pos = dst.reshape(T, TOPK)
    p0 = pos[:, 0] + 0
    p1 = pos[:, 1] + 0

    nl = jnp.asarray(n_live, jnp.int32).reshape(1)

    xs = _gather_rows()(xf, row_token)
    ys = _ffn(te, nl, xs, gate_tiles, W1, W3, W2)
    outf = _combine_rows()(ys, p0, p1)

    return outf.reshape(b, s, d), jnp.asarray(0.0, x.dtype)


# speedup vs baseline: 1.4466x; 1.4466x over previous
import functools
import jax
import jax.numpy as jnp
from jax import lax
from jax.experimental import pallas as pl
from jax.experimental.pallas import tpu as pltpu
from jax.experimental.pallas import tpu_sc as plsc

D_MODEL = 768
D_FF = 2048
NE = 8
TOPK = 2
T = 2048
NP = T * TOPK
BT = 256
N_TILES = NP // BT + NE
N_MAX = N_TILES * BT
NW = 32
GPW = N_MAX // NW
GCH = 64
TPW = T // NW
CCH = 32


def _ffn_tile(te_ref, nl_ref, xs_ref, g_ref, w1_ref, w3_ref, w2_ref, out_ref):
    i = pl.program_id(0)

    @pl.when(i < nl_ref[0])
    def _():
        xv = xs_ref[...]
        h = jnp.dot(xv, w1_ref[0], preferred_element_type=jnp.float32)
        g = jnp.dot(xv, w3_ref[0], preferred_element_type=jnp.float32)
        a = (h * jax.nn.sigmoid(h)) * g
        y = jnp.dot(a, w2_ref[0], preferred_element_type=jnp.float32)
        out_ref[...] = y * g_ref[0, 0][:, None]


def _ffn(te, nl, xs, gate_tiles, W1, W3, W2):
    grid_spec = pltpu.PrefetchScalarGridSpec(
        num_scalar_prefetch=2,
        grid=(N_TILES,),
        in_specs=[
            pl.BlockSpec((BT, D_MODEL), lambda i, te, nl: (i, 0)),
            pl.BlockSpec((1, 1, BT), lambda i, te, nl: (i, 0, 0)),
            pl.BlockSpec((1, D_MODEL, D_FF), lambda i, te, nl: (te[i], 0, 0)),
            pl.BlockSpec((1, D_MODEL, D_FF), lambda i, te, nl: (te[i], 0, 0)),
            pl.BlockSpec((1, D_FF, D_MODEL), lambda i, te, nl: (te[i], 0, 0)),
        ],
        out_specs=pl.BlockSpec((BT, D_MODEL), lambda i, te, nl: (i, 0)),
    )
    return pl.pallas_call(
        _ffn_tile,
        grid_spec=grid_spec,
        out_shape=jax.ShapeDtypeStruct((N_MAX, D_MODEL), jnp.float32),
        compiler_params=pltpu.CompilerParams(
            dimension_semantics=("arbitrary",)),
    )(te, nl, xs, gate_tiles, W1, W3, W2)


@functools.cache
def _sc_mesh():
    return plsc.VectorSubcoreMesh(
        core_axis_name="c", subcore_axis_name="s", num_cores=2)


def _gather_rows_body(x_hbm, idx_hbm, out_hbm, idx_v, rows_a, rows_b, sem_a, sem_b):
    wid = lax.axis_index("s") * 2 + lax.axis_index("c")
    base = wid * GPW
    pltpu.sync_copy(idx_hbm.at[pl.ds(base, GPW)], idx_v)
    bufs = (rows_a, rows_b)
    sems = (sem_a, sem_b)
    nch = GPW // GCH
    cps = [None, None]
    cps[0] = pltpu.async_copy(
        x_hbm.at[idx_v.at[pl.ds(0, GCH)]], rows_a, sem_a)
    for c in range(nch):
        if c + 1 < nch:
            cps[(c + 1) % 2] = pltpu.async_copy(
                x_hbm.at[idx_v.at[pl.ds((c + 1) * GCH, GCH)]],
                bufs[(c + 1) % 2], sems[(c + 1) % 2])
        cps[c % 2].wait()
        pltpu.sync_copy(bufs[c % 2], out_hbm.at[pl.ds(base + c * GCH, GCH)])


@functools.cache
def _gather_rows():
    return functools.partial(
        pl.kernel, mesh=_sc_mesh(),
        out_type=jax.ShapeDtypeStruct((N_MAX, D_MODEL), jnp.float32),
        scratch_types=[
            pltpu.VMEM((GPW,), jnp.int32),
            pltpu.VMEM((GCH, D_MODEL), jnp.float32),
            pltpu.VMEM((GCH, D_MODEL), jnp.float32),
            pltpu.SemaphoreType.DMA,
            pltpu.SemaphoreType.DMA,
        ],
    )(_gather_rows_body)


def _combine_rows_body(ys_hbm, p0_hbm, p1_hbm, out_hbm, i0_v, i1_v, r0_v, r1_v, s0, s1):
    wid = lax.axis_index("s") * 2 + lax.axis_index("c")
    base = wid * TPW

    def chunk(c, carry):
        off = base + c * CCH
        pltpu.sync_copy(p0_hbm.at[pl.ds(off, CCH)], i0_v)
        pltpu.sync_copy(p1_hbm.at[pl.ds(off, CCH)], i1_v)
        cp0 = pltpu.async_copy(ys_hbm.at[i0_v], r0_v, s0)
        cp1 = pltpu.async_copy(ys_hbm.at[i1_v], r1_v, s1)
        cp0.wait()
        cp1.wait()

        def row(i, rc):
            for j in range(D_MODEL // 16):
                sl = pl.ds(j * 16, 16)
                r0_v[i, sl] = r0_v[i, sl] + r1_v[i, sl]
            return rc

        lax.fori_loop(0, CCH, row, 0)
        pltpu.sync_copy(r0_v, out_hbm.at[pl.ds(off, CCH)])
        return carry

    lax.fori_loop(0, TPW // CCH, chunk, 0)


@functools.cache
def _combine_rows():
    return functools.partial(
        pl.kernel, mesh=_sc_mesh(),
        out_type=jax.ShapeDtypeStruct((T, D_MODEL), jnp.float32),
        scratch_types=[
            pltpu.VMEM((CCH,), jnp.int32),
            pltpu.VMEM((CCH,), jnp.int32),
            pltpu.VMEM((CCH, D_MODEL), jnp.float32),
            pltpu.VMEM((CCH, D_MODEL), jnp.float32),
            pltpu.SemaphoreType.DMA,
            pltpu.SemaphoreType.DMA,
        ],
    )(_combine_rows_body)


def kernel(x, Wg, W1, W3, W2):
    b, s, d = x.shape
    xf = x.reshape(-1, d)

    logits = xf @ Wg
    gates = jax.nn.softmax(logits, axis=-1)
    tg, ti = lax.top_k(gates, TOPK)
    tg = tg / jnp.sum(tg, axis=-1, keepdims=True)

    eid = ti.reshape(-1).astype(jnp.int32)
    oh = (eid[:, None] == jnp.arange(NE, dtype=jnp.int32)[None, :]).astype(jnp.int32)
    within = jnp.cumsum(oh, axis=0) - oh
    rank = jnp.take_along_axis(within, eid[:, None], axis=1)[:, 0]
    counts = jnp.sum(oh, axis=0)
    padded = ((counts + BT - 1) // BT) * BT
    cumpad = jnp.cumsum(padded)
    offs = cumpad - padded
    dst = offs[eid] + rank
    total = cumpad[-1]
    n_live = total // BT

    tile_starts = jnp.arange(N_TILES, dtype=jnp.int32) * BT
    te = jnp.searchsorted(cumpad, tile_starts, side="right").astype(jnp.int32)
    te_lastlive = jnp.take(te, jnp.maximum(n_live - 1, 0))
    te = jnp.where(tile_starts < total, te, te_lastlive)

    row_token = (jnp.arange(N_MAX, dtype=jnp.int32) % T).at[dst].set(
        jnp.arange(NP, dtype=jnp.int32) // TOPK)
    row_gate = jnp.zeros((N_MAX,), jnp.float32).at[dst].set(tg.reshape(-1))
    gate_tiles = row_gate.reshape(N_TILES, 1, BT)
    pos = dst.reshape(T, TOPK)
    p0 = pos[:, 0] + 0
    p1 = pos[:, 1] + 0

    nl = jnp.asarray(n_live, jnp.int32).reshape(1)

    xs = _gather_rows()(xf, row_token)
    ys = _ffn(te, nl, xs, gate_tiles, W1, W3, W2)
    outf = _combine_rows()(ys, p0, p1)

    return outf.reshape(b, s, d), jnp.asarray(0.0, x.dtype)
